# baseline (speedup 1.0000x reference)
import functools

import jax
import jax.numpy as jnp
from jax import lax
from jax.experimental import pallas as pl
from jax.experimental.pallas import tpu as pltpu
from jax.experimental.pallas import tpu_sc as plsc

BATCH = 16384
DIM = 128
TBL = 1000000

_INFO = plsc.get_sparse_core_info()
_NC = _INFO.num_cores
_NS = _INFO.num_subcores
_NW = _NC * _NS
_BPW = BATCH // _NW
_CHUNK = 128
_NCHUNK = _BPW // _CHUNK

_CBLK = 8192
_MVBLK = 4096


def _conv_body(bu_ref, bi_ref, obu_ref, obi_ref):
    obu_ref[:] = bu_ref[:, 0]
    obi_ref[:] = bi_ref[:, 0]


def _mv_body(w_ref, b_ref, f_ref, o_ref):
    o_ref[:] = jnp.sum(f_ref[:, :] * w_ref[:, :], axis=1) + b_ref[0]


def _sc_body(bu_hbm, bi_hbm, uid_hbm, iid_hbm, lin_hbm, out_hbm,
             uidx_v, iidx_v, bu_v, bi_v, lin_v, sem_u, sem_i):
    wid = lax.axis_index("s") * _NC + lax.axis_index("c")
    base = wid * _BPW
    pltpu.sync_copy(uid_hbm.at[pl.ds(base, _BPW)], uidx_v)
    pltpu.sync_copy(iid_hbm.at[pl.ds(base, _BPW)], iidx_v)
    pltpu.sync_copy(lin_hbm.at[pl.ds(base, _BPW)], lin_v)
    copies = []
    for j in range(_NCHUNK):
        sl = pl.ds(j * _CHUNK, _CHUNK)
        copies.append(pltpu.async_copy(bu_hbm.at[uidx_v.at[sl]], bu_v.at[sl], sem_u))
        copies.append(pltpu.async_copy(bi_hbm.at[iidx_v.at[sl]], bi_v.at[sl], sem_i))
    for c in copies:
        c.wait()
    for j in range(_BPW // 16):
        sl = pl.ds(j * 16, 16)
        bu_v[sl] = bu_v[sl] + bi_v[sl] + lin_v[sl]
    pltpu.sync_copy(bu_v, out_hbm.at[pl.ds(base, _BPW)])


def _sc_call(bu_lin, bi_lin, uid, iid, lin):
    mesh = plsc.VectorSubcoreMesh(core_axis_name="c", subcore_axis_name="s")
    fn = functools.partial(
        pl.kernel,
        mesh=mesh,
        out_type=jax.ShapeDtypeStruct((BATCH,), jnp.float32),
        scratch_types=[
            pltpu.VMEM((_BPW,), jnp.int32),
            pltpu.VMEM((_BPW,), jnp.int32),
            pltpu.VMEM((_BPW,), jnp.float32),
            pltpu.VMEM((_BPW,), jnp.float32),
            pltpu.VMEM((_BPW,), jnp.float32),
            pltpu.SemaphoreType.DMA,
            pltpu.SemaphoreType.DMA,
        ],
    )(_sc_body)
    return fn(bu_lin, bi_lin, uid, iid, lin)


def kernel(feature, user_id, item_id, fc_w, fc_b, b_users, b_items):
    uid = user_id.astype(jnp.int32)
    iid = item_id.astype(jnp.int32)

    bu_lin, bi_lin = pl.pallas_call(
        _conv_body,
        grid=((TBL + _CBLK - 1) // _CBLK,),
        in_specs=[
            pl.BlockSpec((_CBLK, 1), lambda i: (i, 0)),
            pl.BlockSpec((_CBLK, 1), lambda i: (i, 0)),
        ],
        out_specs=[
            pl.BlockSpec((_CBLK,), lambda i: (i,)),
            pl.BlockSpec((_CBLK,), lambda i: (i,)),
        ],
        out_shape=[
            jax.ShapeDtypeStruct((TBL,), jnp.float32),
            jax.ShapeDtypeStruct((TBL,), jnp.float32),
        ],
    )(b_users, b_items)

    lin = pl.pallas_call(
        _mv_body,
        grid=(BATCH // _MVBLK,),
        in_specs=[
            pl.BlockSpec((1, DIM), lambda i: (0, 0)),
            pl.BlockSpec(memory_space=pltpu.SMEM),
            pl.BlockSpec((_MVBLK, DIM), lambda i: (i, 0)),
        ],
        out_specs=pl.BlockSpec((_MVBLK,), lambda i: (i,)),
        out_shape=jax.ShapeDtypeStruct((BATCH,), jnp.float32),
    )(fc_w, fc_b, feature)

    out1d = _sc_call(bu_lin, bi_lin, uid, iid, lin)
    return out1d.reshape(BATCH, 1)


# speedup vs baseline: 25.0736x; 25.0736x over previous
import functools

import jax
import jax.numpy as jnp
from jax import lax
from jax.experimental import pallas as pl
from jax.experimental.pallas import tpu as pltpu
from jax.experimental.pallas import tpu_sc as plsc

BATCH = 16384
DIM = 128
TBL = 1000000
TBLP = 8192 * 128

_INFO = plsc.get_sparse_core_info()
_NC = _INFO.num_cores
_NS = _INFO.num_subcores
_NW = _NC * _NS
_BPW = BATCH // _NW
_CHUNK = 128
_NCHUNK = _BPW // _CHUNK

_CBLK = 65536
_MVBLK = 4096


def _pad_body(bu_ref, bi_ref, obu_ref, obi_ref):
    obu_ref[:, :] = bu_ref[:, :]
    obi_ref[:, :] = bi_ref[:, :]


def _mv_body(w_ref, b_ref, f_ref, o_ref):
    o_ref[:] = jnp.sum(f_ref[:, :] * w_ref[:, :], axis=1) + b_ref[0]


def _sc_body(ut_hbm, it_hbm, uid_hbm, iid_hbm, lin_hbm, out_hbm,
             uidx_v, iidx_v, urow_v, irow_v, lin_v, acc_v,
             rows_u, rows_i, sem_u, sem_i):
    wid = lax.axis_index("s") * _NC + lax.axis_index("c")
    base = wid * _BPW
    pltpu.sync_copy(uid_hbm.at[pl.ds(base, _BPW)], uidx_v)
    pltpu.sync_copy(iid_hbm.at[pl.ds(base, _BPW)], iidx_v)
    pltpu.sync_copy(lin_hbm.at[pl.ds(base, _BPW)], lin_v)

    seven = jnp.full((16,), 7, jnp.int32)
    low7 = jnp.full((16,), 127, jnp.int32)
    lane = lax.iota(jnp.int32, 16)
    for s in range(_BPW // 16):
        sl = pl.ds(s * 16, 16)
        urow_v[sl] = lax.shift_right_logical(uidx_v[sl], seven)
        irow_v[sl] = lax.shift_right_logical(iidx_v[sl], seven)

    for j in range(_NCHUNK):
        slc = pl.ds(j * _CHUNK, _CHUNK)
        cu = pltpu.async_copy(ut_hbm.at[urow_v.at[slc]], rows_u, sem_u)
        ci = pltpu.async_copy(it_hbm.at[irow_v.at[slc]], rows_i, sem_i)
        cu.wait()
        ci.wait()
        for k in range(_CHUNK // 16):
            sl_abs = pl.ds(j * _CHUNK + k * 16, 16)
            rid = jnp.full((16,), k * 16, jnp.int32) + lane
            ucol = uidx_v[sl_abs] & low7
            icol = iidx_v[sl_abs] & low7
            vu = plsc.load_gather(rows_u, [rid, ucol])
            vi = plsc.load_gather(rows_i, [rid, icol])
            acc_v[sl_abs] = vu + vi + lin_v[sl_abs]

    pltpu.sync_copy(acc_v, out_hbm.at[pl.ds(base, _BPW)])


def _sc_call(ut2, it2, uid, iid, lin):
    mesh = plsc.VectorSubcoreMesh(core_axis_name="c", subcore_axis_name="s")
    fn = functools.partial(
        pl.kernel,
        mesh=mesh,
        compiler_params=pltpu.CompilerParams(needs_layout_passes=False),
        out_type=jax.ShapeDtypeStruct((BATCH,), jnp.float32),
        scratch_types=[
            pltpu.VMEM((_BPW,), jnp.int32),
            pltpu.VMEM((_BPW,), jnp.int32),
            pltpu.VMEM((_BPW,), jnp.int32),
            pltpu.VMEM((_BPW,), jnp.int32),
            pltpu.VMEM((_BPW,), jnp.float32),
            pltpu.VMEM((_BPW,), jnp.float32),
            pltpu.VMEM((_CHUNK, 128), jnp.float32),
            pltpu.VMEM((_CHUNK, 128), jnp.float32),
            pltpu.SemaphoreType.DMA,
            pltpu.SemaphoreType.DMA,
        ],
    )(_sc_body)
    return fn(ut2, it2, uid, iid, lin)


def kernel(feature, user_id, item_id, fc_w, fc_b, b_users, b_items):
    uid = user_id.astype(jnp.int32)
    iid = item_id.astype(jnp.int32)

    bu1 = b_users.reshape(1, TBL)
    bi1 = b_items.reshape(1, TBL)
    oup, oip = pl.pallas_call(
        _pad_body,
        grid=(TBLP // _CBLK,),
        in_specs=[
            pl.BlockSpec((1, _CBLK), lambda i: (0, i)),
            pl.BlockSpec((1, _CBLK), lambda i: (0, i)),
        ],
        out_specs=[
            pl.BlockSpec((1, _CBLK), lambda i: (0, i)),
            pl.BlockSpec((1, _CBLK), lambda i: (0, i)),
        ],
        out_shape=[
            jax.ShapeDtypeStruct((1, TBLP), jnp.float32),
            jax.ShapeDtypeStruct((1, TBLP), jnp.float32),
        ],
    )(bu1, bi1)
    ut2 = oup.reshape(TBLP // 128, 128)
    it2 = oip.reshape(TBLP // 128, 128)

    lin = pl.pallas_call(
        _mv_body,
        grid=(BATCH // _MVBLK,),
        in_specs=[
            pl.BlockSpec((1, DIM), lambda i: (0, 0)),
            pl.BlockSpec(memory_space=pltpu.SMEM),
            pl.BlockSpec((_MVBLK, DIM), lambda i: (i, 0)),
        ],
        out_specs=pl.BlockSpec((_MVBLK,), lambda i: (i,)),
        out_shape=jax.ShapeDtypeStruct((BATCH,), jnp.float32),
    )(fc_w, fc_b, feature)

    out1d = _sc_call(ut2, it2, uid, iid, lin)
    return out1d.reshape(BATCH, 1)


# speedup vs baseline: 34.1188x; 1.3607x over previous
import functools

import jax
import jax.numpy as jnp
from jax import lax
from jax.experimental import pallas as pl
from jax.experimental.pallas import tpu as pltpu
from jax.experimental.pallas import tpu_sc as plsc

BATCH = 16384
DIM = 128
TBL = 1000000
TBLP = 8192 * 128

_INFO = plsc.get_sparse_core_info()
_NC = _INFO.num_cores
_NS = _INFO.num_subcores
_NW = _NC * _NS
_BPW = BATCH // _NW
_CHUNK = 128
_NCHUNK = _BPW // _CHUNK

_CBLK = 131072
_MVBLK = 4096


def _pad_body(bu_ref, bi_ref, obu_ref, obi_ref):
    obu_ref[:, :] = bu_ref[:, :]
    obi_ref[:, :] = bi_ref[:, :]


def _mv_body(w_ref, b_ref, f_ref, o_ref):
    o_ref[:] = jnp.sum(f_ref[:, :] * w_ref[:, :], axis=1) + b_ref[0]


def _comb_body(lin_ref, g_ref, o_ref):
    o_ref[:] = lin_ref[:] + g_ref[:]


def _sc_body(ut_hbm, it_hbm, uid_hbm, iid_hbm, out_hbm,
             uidx_v, iidx_v, urow_v, irow_v, acc_v,
             rows_u0, rows_u1, rows_i0, rows_i1,
             sem_a, sem_b, sem_u0, sem_u1, sem_i0, sem_i1):
    wid = lax.axis_index("s") * _NC + lax.axis_index("c")
    base = wid * _BPW
    cu = pltpu.async_copy(uid_hbm.at[pl.ds(base, _BPW)], uidx_v, sem_a)
    ci = pltpu.async_copy(iid_hbm.at[pl.ds(base, _BPW)], iidx_v, sem_b)
    cu.wait()
    ci.wait()

    seven = jnp.full((16,), 7, jnp.int32)
    low7 = jnp.full((16,), 127, jnp.int32)
    lane = lax.iota(jnp.int32, 16)
    for s in range(_BPW // 16):
        sl = pl.ds(s * 16, 16)
        urow_v[sl] = lax.shift_right_logical(uidx_v[sl], seven)
        irow_v[sl] = lax.shift_right_logical(iidx_v[sl], seven)

    bufs_u = (rows_u0, rows_u1)
    bufs_i = (rows_i0, rows_i1)
    sems_u = (sem_u0, sem_u1)
    sems_i = (sem_i0, sem_i1)

    def fire(j):
        slc = pl.ds(j * _CHUNK, _CHUNK)
        hu = pltpu.async_copy(ut_hbm.at[urow_v.at[slc]], bufs_u[j % 2],
                              sems_u[j % 2])
        hi = pltpu.async_copy(it_hbm.at[irow_v.at[slc]], bufs_i[j % 2],
                              sems_i[j % 2])
        return hu, hi

    handles = [None] * _NCHUNK
    for j in range(min(2, _NCHUNK)):
        handles[j] = fire(j)
    for j in range(_NCHUNK):
        hu, hi = handles[j]
        hu.wait()
        hi.wait()
        ru, ri = bufs_u[j % 2], bufs_i[j % 2]
        for k in range(_CHUNK // 16):
            sl_abs = pl.ds(j * _CHUNK + k * 16, 16)
            rid = jnp.full((16,), k * 16, jnp.int32) + lane
            ucol = uidx_v[sl_abs] & low7
            icol = iidx_v[sl_abs] & low7
            vu = plsc.load_gather(ru, [rid, ucol])
            vi = plsc.load_gather(ri, [rid, icol])
            acc_v[sl_abs] = vu + vi
        if j + 2 < _NCHUNK:
            handles[j + 2] = fire(j + 2)

    pltpu.sync_copy(acc_v, out_hbm.at[pl.ds(base, _BPW)])


def _sc_call(ut2, it2, uid, iid):
    mesh = plsc.VectorSubcoreMesh(core_axis_name="c", subcore_axis_name="s")
    fn = functools.partial(
        pl.kernel,
        mesh=mesh,
        compiler_params=pltpu.CompilerParams(needs_layout_passes=False),
        out_type=jax.ShapeDtypeStruct((BATCH,), jnp.float32),
        scratch_types=[
            pltpu.VMEM((_BPW,), jnp.int32),
            pltpu.VMEM((_BPW,), jnp.int32),
            pltpu.VMEM((_BPW,), jnp.int32),
            pltpu.VMEM((_BPW,), jnp.int32),
            pltpu.VMEM((_BPW,), jnp.float32),
            pltpu.VMEM((_CHUNK, 128), jnp.float32),
            pltpu.VMEM((_CHUNK, 128), jnp.float32),
            pltpu.VMEM((_CHUNK, 128), jnp.float32),
            pltpu.VMEM((_CHUNK, 128), jnp.float32),
            pltpu.SemaphoreType.DMA,
            pltpu.SemaphoreType.DMA,
            pltpu.SemaphoreType.DMA,
            pltpu.SemaphoreType.DMA,
            pltpu.SemaphoreType.DMA,
            pltpu.SemaphoreType.DMA,
        ],
    )(_sc_body)
    return fn(ut2, it2, uid, iid)


def kernel(feature, user_id, item_id, fc_w, fc_b, b_users, b_items):
    uid = user_id.astype(jnp.int32)
    iid = item_id.astype(jnp.int32)

    bu1 = b_users.reshape(1, TBL)
    bi1 = b_items.reshape(1, TBL)
    oup, oip = pl.pallas_call(
        _pad_body,
        grid=(TBLP // _CBLK,),
        in_specs=[
            pl.BlockSpec((1, _CBLK), lambda i: (0, i)),
            pl.BlockSpec((1, _CBLK), lambda i: (0, i)),
        ],
        out_specs=[
            pl.BlockSpec((1, _CBLK), lambda i: (0, i)),
            pl.BlockSpec((1, _CBLK), lambda i: (0, i)),
        ],
        out_shape=[
            jax.ShapeDtypeStruct((1, TBLP), jnp.float32),
            jax.ShapeDtypeStruct((1, TBLP), jnp.float32),
        ],
    )(bu1, bi1)
    ut2 = oup.reshape(TBLP // 128, 128)
    it2 = oip.reshape(TBLP // 128, 128)

    g = _sc_call(ut2, it2, uid, iid)

    lin = pl.pallas_call(
        _mv_body,
        grid=(BATCH // _MVBLK,),
        in_specs=[
            pl.BlockSpec((1, DIM), lambda i: (0, 0)),
            pl.BlockSpec(memory_space=pltpu.SMEM),
            pl.BlockSpec((_MVBLK, DIM), lambda i: (i, 0)),
        ],
        out_specs=pl.BlockSpec((_MVBLK,), lambda i: (i,)),
        out_shape=jax.ShapeDtypeStruct((BATCH,), jnp.float32),
    )(fc_w, fc_b, feature)

    out1d = pl.pallas_call(
        _comb_body,
        in_specs=[
            pl.BlockSpec((BATCH,), lambda: (0,)),
            pl.BlockSpec((BATCH,), lambda: (0,)),
        ],
        out_specs=pl.BlockSpec((BATCH,), lambda: (0,)),
        out_shape=jax.ShapeDtypeStruct((BATCH,), jnp.float32),
    )(lin, g)
    return out1d.reshape(BATCH, 1)


# speedup vs baseline: 34.3314x; 1.0062x over previous
import functools

import jax
import jax.numpy as jnp
from jax import lax
from jax.experimental import pallas as pl
from jax.experimental.pallas import tpu as pltpu
from jax.experimental.pallas import tpu_sc as plsc

BATCH = 16384
DIM = 128
TBL = 1000000
TBLP = 8192 * 128

_INFO = plsc.get_sparse_core_info()
_NC = _INFO.num_cores
_NS = _INFO.num_subcores
_NW = _NC * _NS
_BPW = BATCH // _NW
_CHUNK = 64
_NBUF = 4
_NCHUNK = _BPW // _CHUNK

_CBLK = 262144
_MVBLK = 2048


def _pad_body(bu_ref, bi_ref, obu_ref, obi_ref):
    obu_ref[:, :] = bu_ref[:, :]
    obi_ref[:, :] = bi_ref[:, :]


def _mv_body(w_ref, b_ref, f_ref, o_ref):
    o_ref[:] = jnp.sum(f_ref[:, :] * w_ref[:, :], axis=1) + b_ref[0]


def _comb_body(lin_ref, g_ref, o_ref):
    o_ref[:] = lin_ref[:] + g_ref[:]


def _sc_body(ut_hbm, it_hbm, uid_hbm, iid_hbm, out_hbm,
             uidx_v, iidx_v, urow_v, irow_v, acc_v,
             rows_u0, rows_u1, rows_u2, rows_u3,
             rows_i0, rows_i1, rows_i2, rows_i3,
             sem_a, sem_b, sem_u0, sem_u1, sem_u2, sem_u3,
             sem_i0, sem_i1, sem_i2, sem_i3):
    wid = lax.axis_index("s") * _NC + lax.axis_index("c")
    base = wid * _BPW
    cu = pltpu.async_copy(uid_hbm.at[pl.ds(base, _BPW)], uidx_v, sem_a)
    ci = pltpu.async_copy(iid_hbm.at[pl.ds(base, _BPW)], iidx_v, sem_b)
    cu.wait()
    ci.wait()

    seven = jnp.full((16,), 7, jnp.int32)
    low7 = jnp.full((16,), 127, jnp.int32)
    lane = lax.iota(jnp.int32, 16)
    for s in range(_BPW // 16):
        sl = pl.ds(s * 16, 16)
        urow_v[sl] = lax.shift_right_logical(uidx_v[sl], seven)
        irow_v[sl] = lax.shift_right_logical(iidx_v[sl], seven)

    bufs_u = (rows_u0, rows_u1, rows_u2, rows_u3)
    bufs_i = (rows_i0, rows_i1, rows_i2, rows_i3)
    sems_u = (sem_u0, sem_u1, sem_u2, sem_u3)
    sems_i = (sem_i0, sem_i1, sem_i2, sem_i3)

    def fire(j):
        slc = pl.ds(j * _CHUNK, _CHUNK)
        hu = pltpu.async_copy(ut_hbm.at[urow_v.at[slc]], bufs_u[j % _NBUF],
                              sems_u[j % _NBUF])
        hi = pltpu.async_copy(it_hbm.at[irow_v.at[slc]], bufs_i[j % _NBUF],
                              sems_i[j % _NBUF])
        return hu, hi

    handles = [None] * _NCHUNK
    for j in range(min(_NBUF, _NCHUNK)):
        handles[j] = fire(j)
    for j in range(_NCHUNK):
        hu, hi = handles[j]
        hu.wait()
        hi.wait()
        ru, ri = bufs_u[j % _NBUF], bufs_i[j % _NBUF]
        for k in range(_CHUNK // 16):
            sl_abs = pl.ds(j * _CHUNK + k * 16, 16)
            rid = jnp.full((16,), k * 16, jnp.int32) + lane
            ucol = uidx_v[sl_abs] & low7
            icol = iidx_v[sl_abs] & low7
            vu = plsc.load_gather(ru, [rid, ucol])
            vi = plsc.load_gather(ri, [rid, icol])
            acc_v[sl_abs] = vu + vi
        if j + _NBUF < _NCHUNK:
            handles[j + _NBUF] = fire(j + _NBUF)

    pltpu.sync_copy(acc_v, out_hbm.at[pl.ds(base, _BPW)])


def _sc_call(ut2, it2, uid, iid):
    mesh = plsc.VectorSubcoreMesh(core_axis_name="c", subcore_axis_name="s")
    fn = functools.partial(
        pl.kernel,
        mesh=mesh,
        compiler_params=pltpu.CompilerParams(needs_layout_passes=False),
        out_type=jax.ShapeDtypeStruct((BATCH,), jnp.float32),
        scratch_types=[
            pltpu.VMEM((_BPW,), jnp.int32),
            pltpu.VMEM((_BPW,), jnp.int32),
            pltpu.VMEM((_BPW,), jnp.int32),
            pltpu.VMEM((_BPW,), jnp.int32),
            pltpu.VMEM((_BPW,), jnp.float32),
            pltpu.VMEM((_CHUNK, 128), jnp.float32),
            pltpu.VMEM((_CHUNK, 128), jnp.float32),
            pltpu.VMEM((_CHUNK, 128), jnp.float32),
            pltpu.VMEM((_CHUNK, 128), jnp.float32),
            pltpu.VMEM((_CHUNK, 128), jnp.float32),
            pltpu.VMEM((_CHUNK, 128), jnp.float32),
            pltpu.VMEM((_CHUNK, 128), jnp.float32),
            pltpu.VMEM((_CHUNK, 128), jnp.float32),
            pltpu.SemaphoreType.DMA,
            pltpu.SemaphoreType.DMA,
            pltpu.SemaphoreType.DMA,
            pltpu.SemaphoreType.DMA,
            pltpu.SemaphoreType.DMA,
            pltpu.SemaphoreType.DMA,
            pltpu.SemaphoreType.DMA,
            pltpu.SemaphoreType.DMA,
            pltpu.SemaphoreType.DMA,
            pltpu.SemaphoreType.DMA,
        ],
    )(_sc_body)
    return fn(ut2, it2, uid, iid)


def kernel(feature, user_id, item_id, fc_w, fc_b, b_users, b_items):
    uid = user_id.astype(jnp.int32)
    iid = item_id.astype(jnp.int32)

    bu1 = b_users.reshape(1, TBL)
    bi1 = b_items.reshape(1, TBL)
    oup, oip = pl.pallas_call(
        _pad_body,
        grid=(TBLP // _CBLK,),
        in_specs=[
            pl.BlockSpec((1, _CBLK), lambda i: (0, i)),
            pl.BlockSpec((1, _CBLK), lambda i: (0, i)),
        ],
        out_specs=[
            pl.BlockSpec((1, _CBLK), lambda i: (0, i)),
            pl.BlockSpec((1, _CBLK), lambda i: (0, i)),
        ],
        out_shape=[
            jax.ShapeDtypeStruct((1, TBLP), jnp.float32),
            jax.ShapeDtypeStruct((1, TBLP), jnp.float32),
        ],
    )(bu1, bi1)
    ut2 = oup.reshape(TBLP // 128, 128)
    it2 = oip.reshape(TBLP // 128, 128)

    g = _sc_call(ut2, it2, uid, iid)

    lin = pl.pallas_call(
        _mv_body,
        grid=(BATCH // _MVBLK,),
        in_specs=[
            pl.BlockSpec((1, DIM), lambda i: (0, 0)),
            pl.BlockSpec(memory_space=pltpu.SMEM),
            pl.BlockSpec((_MVBLK, DIM), lambda i: (i, 0)),
        ],
        out_specs=pl.BlockSpec((_MVBLK,), lambda i: (i,)),
        out_shape=jax.ShapeDtypeStruct((BATCH,), jnp.float32),
    )(fc_w, fc_b, feature)

    out1d = pl.pallas_call(
        _comb_body,
        in_specs=[
            pl.BlockSpec((BATCH,), lambda: (0,)),
            pl.BlockSpec((BATCH,), lambda: (0,)),
        ],
        out_specs=pl.BlockSpec((BATCH,), lambda: (0,)),
        out_shape=jax.ShapeDtypeStruct((BATCH,), jnp.float32),
    )(lin, g)
    return out1d.reshape(BATCH, 1)


# speedup vs baseline: 37.6483x; 1.0966x over previous
import functools

import jax
import jax.numpy as jnp
from jax import lax
from jax.experimental import pallas as pl
from jax.experimental.pallas import tpu as pltpu
from jax.experimental.pallas import tpu_sc as plsc

BATCH = 16384
DIM = 128
TBL = 1000000
TBLP = 8192 * 128

_INFO = plsc.get_sparse_core_info()
_NC = _INFO.num_cores
_NS = _INFO.num_subcores
_NW = _NC * _NS
_BPW = BATCH // _NW
_CHUNK = 64
_NBUF = 4
_NCHUNK = _BPW // _CHUNK

_CBLK = 524288
_MVBLK = 4096


def _pad_body(bu_ref, bi_ref, obu_ref, obi_ref):
    obu_ref[:, :] = bu_ref[:, :]
    obi_ref[:, :] = bi_ref[:, :]


def _mv_body(w_ref, b_ref, f_ref, o_ref):
    o_ref[:] = jnp.sum(f_ref[:, :] * w_ref[:, :], axis=1) + b_ref[0]


def _comb_body(lin_ref, g_ref, o_ref):
    o_ref[:] = lin_ref[:] + g_ref[:]


def _sc_body(ut_hbm, it_hbm, uid_hbm, iid_hbm, out_hbm,
             uidx_v, iidx_v, urow_v, irow_v, acc_v,
             rows_u0, rows_u1, rows_u2, rows_u3,
             rows_i0, rows_i1, rows_i2, rows_i3,
             sem_a, sem_b, sem_u0, sem_u1, sem_u2, sem_u3,
             sem_i0, sem_i1, sem_i2, sem_i3):
    wid = lax.axis_index("s") * _NC + lax.axis_index("c")
    base = wid * _BPW
    cu = pltpu.async_copy(uid_hbm.at[pl.ds(base, _BPW)], uidx_v, sem_a)
    ci = pltpu.async_copy(iid_hbm.at[pl.ds(base, _BPW)], iidx_v, sem_b)
    cu.wait()
    ci.wait()

    seven = jnp.full((16,), 7, jnp.int32)
    low7 = jnp.full((16,), 127, jnp.int32)
    lane = lax.iota(jnp.int32, 16)
    for s in range(_BPW // 16):
        sl = pl.ds(s * 16, 16)
        urow_v[sl] = lax.shift_right_logical(uidx_v[sl], seven)
        irow_v[sl] = lax.shift_right_logical(iidx_v[sl], seven)

    bufs_u = (rows_u0, rows_u1, rows_u2, rows_u3)
    bufs_i = (rows_i0, rows_i1, rows_i2, rows_i3)
    sems_u = (sem_u0, sem_u1, sem_u2, sem_u3)
    sems_i = (sem_i0, sem_i1, sem_i2, sem_i3)

    def fire(j):
        slc = pl.ds(j * _CHUNK, _CHUNK)
        hu = pltpu.async_copy(ut_hbm.at[urow_v.at[slc]], bufs_u[j % _NBUF],
                              sems_u[j % _NBUF])
        hi = pltpu.async_copy(it_hbm.at[irow_v.at[slc]], bufs_i[j % _NBUF],
                              sems_i[j % _NBUF])
        return hu, hi

    handles = [None] * _NCHUNK
    for j in range(min(_NBUF, _NCHUNK)):
        handles[j] = fire(j)
    for j in range(_NCHUNK):
        hu, hi = handles[j]
        hu.wait()
        hi.wait()
        ru, ri = bufs_u[j % _NBUF], bufs_i[j % _NBUF]
        for k in range(_CHUNK // 16):
            sl_abs = pl.ds(j * _CHUNK + k * 16, 16)
            rid = jnp.full((16,), k * 16, jnp.int32) + lane
            ucol = uidx_v[sl_abs] & low7
            icol = iidx_v[sl_abs] & low7
            vu = plsc.load_gather(ru, [rid, ucol])
            vi = plsc.load_gather(ri, [rid, icol])
            acc_v[sl_abs] = vu + vi
        if j + _NBUF < _NCHUNK:
            handles[j + _NBUF] = fire(j + _NBUF)

    pltpu.sync_copy(acc_v, out_hbm.at[pl.ds(base, _BPW)])


def _sc_call(ut2, it2, uid, iid):
    mesh = plsc.VectorSubcoreMesh(core_axis_name="c", subcore_axis_name="s")
    fn = functools.partial(
        pl.kernel,
        mesh=mesh,
        compiler_params=pltpu.CompilerParams(needs_layout_passes=False),
        out_type=jax.ShapeDtypeStruct((BATCH,), jnp.float32),
        scratch_types=[
            pltpu.VMEM((_BPW,), jnp.int32),
            pltpu.VMEM((_BPW,), jnp.int32),
            pltpu.VMEM((_BPW,), jnp.int32),
            pltpu.VMEM((_BPW,), jnp.int32),
            pltpu.VMEM((_BPW,), jnp.float32),
            pltpu.VMEM((_CHUNK, 128), jnp.float32),
            pltpu.VMEM((_CHUNK, 128), jnp.float32),
            pltpu.VMEM((_CHUNK, 128), jnp.float32),
            pltpu.VMEM((_CHUNK, 128), jnp.float32),
            pltpu.VMEM((_CHUNK, 128), jnp.float32),
            pltpu.VMEM((_CHUNK, 128), jnp.float32),
            pltpu.VMEM((_CHUNK, 128), jnp.float32),
            pltpu.VMEM((_CHUNK, 128), jnp.float32),
            pltpu.SemaphoreType.DMA,
            pltpu.SemaphoreType.DMA,
            pltpu.SemaphoreType.DMA,
            pltpu.SemaphoreType.DMA,
            pltpu.SemaphoreType.DMA,
            pltpu.SemaphoreType.DMA,
            pltpu.SemaphoreType.DMA,
            pltpu.SemaphoreType.DMA,
            pltpu.SemaphoreType.DMA,
            pltpu.SemaphoreType.DMA,
        ],
    )(_sc_body)
    return fn(ut2, it2, uid, iid)


def kernel(feature, user_id, item_id, fc_w, fc_b, b_users, b_items):
    uid = user_id.astype(jnp.int32)
    iid = item_id.astype(jnp.int32)

    bu1 = b_users.reshape(1, TBL)
    bi1 = b_items.reshape(1, TBL)
    oup, oip = pl.pallas_call(
        _pad_body,
        grid=(TBLP // _CBLK,),
        in_specs=[
            pl.BlockSpec((1, _CBLK), lambda i: (0, i)),
            pl.BlockSpec((1, _CBLK), lambda i: (0, i)),
        ],
        out_specs=[
            pl.BlockSpec((1, _CBLK), lambda i: (0, i)),
            pl.BlockSpec((1, _CBLK), lambda i: (0, i)),
        ],
        out_shape=[
            jax.ShapeDtypeStruct((1, TBLP), jnp.float32),
            jax.ShapeDtypeStruct((1, TBLP), jnp.float32),
        ],
    )(bu1, bi1)
    ut2 = oup.reshape(TBLP // 128, 128)
    it2 = oip.reshape(TBLP // 128, 128)

    g = _sc_call(ut2, it2, uid, iid)

    lin = pl.pallas_call(
        _mv_body,
        grid=(BATCH // _MVBLK,),
        in_specs=[
            pl.BlockSpec((1, DIM), lambda i: (0, 0)),
            pl.BlockSpec(memory_space=pltpu.SMEM),
            pl.BlockSpec((_MVBLK, DIM), lambda i: (i, 0)),
        ],
        out_specs=pl.BlockSpec((_MVBLK,), lambda i: (i,)),
        out_shape=jax.ShapeDtypeStruct((BATCH,), jnp.float32),
    )(fc_w, fc_b, feature)

    out1d = pl.pallas_call(
        _comb_body,
        in_specs=[
            pl.BlockSpec((BATCH,), lambda: (0,)),
            pl.BlockSpec((BATCH,), lambda: (0,)),
        ],
        out_specs=pl.BlockSpec((BATCH,), lambda: (0,)),
        out_shape=jax.ShapeDtypeStruct((BATCH,), jnp.float32),
    )(lin, g)
    return out1d.reshape(BATCH, 1)


# speedup vs baseline: 38.3069x; 1.0175x over previous
import functools

import jax
import jax.numpy as jnp
from jax import lax
from jax.experimental import pallas as pl
from jax.experimental.pallas import tpu as pltpu
from jax.experimental.pallas import tpu_sc as plsc

BATCH = 16384
DIM = 128
TBL = 1000000
TBLP = 8192 * 128

_INFO = plsc.get_sparse_core_info()
_NC = _INFO.num_cores
_NS = _INFO.num_subcores
_NW = _NC * _NS
_BPW = BATCH // _NW
_CHUNK = 64
_NBUF = 4
_NCHUNK = _BPW // _CHUNK

_CBLK = 524288
_MVBLK = 8192


def _pad_body(bu_ref, bi_ref, obu_ref, obi_ref):
    obu_ref[:, :] = bu_ref[:, :]
    obi_ref[:, :] = bi_ref[:, :]


def _mv_body(w_ref, b_ref, f_ref, o_ref):
    o_ref[:] = jnp.sum(f_ref[:, :] * w_ref[:, :], axis=1) + b_ref[0]


def _comb_body(lin_ref, g_ref, o_ref):
    o_ref[:] = lin_ref[:] + g_ref[:]


def _sc_body(ut_hbm, it_hbm, uid_hbm, iid_hbm, out_hbm,
             uidx_v, iidx_v, urow_v, irow_v, acc_v,
             rows_u0, rows_u1, rows_u2, rows_u3,
             rows_i0, rows_i1, rows_i2, rows_i3,
             sem_a, sem_b, sem_u0, sem_u1, sem_u2, sem_u3,
             sem_i0, sem_i1, sem_i2, sem_i3):
    wid = lax.axis_index("s") * _NC + lax.axis_index("c")
    base = wid * _BPW
    cu = pltpu.async_copy(uid_hbm.at[pl.ds(base, _BPW)], uidx_v, sem_a)
    ci = pltpu.async_copy(iid_hbm.at[pl.ds(base, _BPW)], iidx_v, sem_b)
    cu.wait()
    ci.wait()

    seven = jnp.full((16,), 7, jnp.int32)
    low7 = jnp.full((16,), 127, jnp.int32)
    lane = lax.iota(jnp.int32, 16)
    for s in range(_BPW // 16):
        sl = pl.ds(s * 16, 16)
        urow_v[sl] = lax.shift_right_logical(uidx_v[sl], seven)
        irow_v[sl] = lax.shift_right_logical(iidx_v[sl], seven)

    bufs_u = (rows_u0, rows_u1, rows_u2, rows_u3)
    bufs_i = (rows_i0, rows_i1, rows_i2, rows_i3)
    sems_u = (sem_u0, sem_u1, sem_u2, sem_u3)
    sems_i = (sem_i0, sem_i1, sem_i2, sem_i3)

    def fire(j):
        slc = pl.ds(j * _CHUNK, _CHUNK)
        hu = pltpu.async_copy(ut_hbm.at[urow_v.at[slc]], bufs_u[j % _NBUF],
                              sems_u[j % _NBUF])
        hi = pltpu.async_copy(it_hbm.at[irow_v.at[slc]], bufs_i[j % _NBUF],
                              sems_i[j % _NBUF])
        return hu, hi

    handles = [None] * _NCHUNK
    for j in range(min(_NBUF, _NCHUNK)):
        handles[j] = fire(j)
    for j in range(_NCHUNK):
        hu, hi = handles[j]
        hu.wait()
        hi.wait()
        ru, ri = bufs_u[j % _NBUF], bufs_i[j % _NBUF]
        for k in range(_CHUNK // 16):
            sl_abs = pl.ds(j * _CHUNK + k * 16, 16)
            rid = jnp.full((16,), k * 16, jnp.int32) + lane
            ucol = uidx_v[sl_abs] & low7
            icol = iidx_v[sl_abs] & low7
            vu = plsc.load_gather(ru, [rid, ucol])
            vi = plsc.load_gather(ri, [rid, icol])
            acc_v[sl_abs] = vu + vi
        if j + _NBUF < _NCHUNK:
            handles[j + _NBUF] = fire(j + _NBUF)

    pltpu.sync_copy(acc_v, out_hbm.at[pl.ds(base, _BPW)])


def _sc_call(ut2, it2, uid, iid):
    mesh = plsc.VectorSubcoreMesh(core_axis_name="c", subcore_axis_name="s")
    fn = functools.partial(
        pl.kernel,
        mesh=mesh,
        compiler_params=pltpu.CompilerParams(needs_layout_passes=False),
        out_type=jax.ShapeDtypeStruct((BATCH,), jnp.float32),
        scratch_types=[
            pltpu.VMEM((_BPW,), jnp.int32),
            pltpu.VMEM((_BPW,), jnp.int32),
            pltpu.VMEM((_BPW,), jnp.int32),
            pltpu.VMEM((_BPW,), jnp.int32),
            pltpu.VMEM((_BPW,), jnp.float32),
            pltpu.VMEM((_CHUNK, 128), jnp.float32),
            pltpu.VMEM((_CHUNK, 128), jnp.float32),
            pltpu.VMEM((_CHUNK, 128), jnp.float32),
            pltpu.VMEM((_CHUNK, 128), jnp.float32),
            pltpu.VMEM((_CHUNK, 128), jnp.float32),
            pltpu.VMEM((_CHUNK, 128), jnp.float32),
            pltpu.VMEM((_CHUNK, 128), jnp.float32),
            pltpu.VMEM((_CHUNK, 128), jnp.float32),
            pltpu.SemaphoreType.DMA,
            pltpu.SemaphoreType.DMA,
            pltpu.SemaphoreType.DMA,
            pltpu.SemaphoreType.DMA,
            pltpu.SemaphoreType.DMA,
            pltpu.SemaphoreType.DMA,
            pltpu.SemaphoreType.DMA,
            pltpu.SemaphoreType.DMA,
            pltpu.SemaphoreType.DMA,
            pltpu.SemaphoreType.DMA,
        ],
    )(_sc_body)
    return fn(ut2, it2, uid, iid)


def kernel(feature, user_id, item_id, fc_w, fc_b, b_users, b_items):
    uid = user_id.astype(jnp.int32)
    iid = item_id.astype(jnp.int32)

    bu1 = b_users.reshape(1, TBL)
    bi1 = b_items.reshape(1, TBL)
    oup, oip = pl.pallas_call(
        _pad_body,
        grid=(TBLP // _CBLK,),
        in_specs=[
            pl.BlockSpec((1, _CBLK), lambda i: (0, i)),
            pl.BlockSpec((1, _CBLK), lambda i: (0, i)),
        ],
        out_specs=[
            pl.BlockSpec((1, _CBLK), lambda i: (0, i)),
            pl.BlockSpec((1, _CBLK), lambda i: (0, i)),
        ],
        out_shape=[
            jax.ShapeDtypeStruct((1, TBLP), jnp.float32),
            jax.ShapeDtypeStruct((1, TBLP), jnp.float32),
        ],
    )(bu1, bi1)
    ut2 = oup.reshape(TBLP // 128, 128)
    it2 = oip.reshape(TBLP // 128, 128)

    g = _sc_call(ut2, it2, uid, iid)

    lin = pl.pallas_call(
        _mv_body,
        grid=(BATCH // _MVBLK,),
        in_specs=[
            pl.BlockSpec((1, DIM), lambda i: (0, 0)),
            pl.BlockSpec(memory_space=pltpu.SMEM),
            pl.BlockSpec((_MVBLK, DIM), lambda i: (i, 0)),
        ],
        out_specs=pl.BlockSpec((_MVBLK,), lambda i: (i,)),
        out_shape=jax.ShapeDtypeStruct((BATCH,), jnp.float32),
    )(fc_w, fc_b, feature)

    out1d = pl.pallas_call(
        _comb_body,
        in_specs=[
            pl.BlockSpec((BATCH,), lambda: (0,)),
            pl.BlockSpec((BATCH,), lambda: (0,)),
        ],
        out_specs=pl.BlockSpec((BATCH,), lambda: (0,)),
        out_shape=jax.ShapeDtypeStruct((BATCH,), jnp.float32),
    )(lin, g)
    return out1d.reshape(BATCH, 1)
